# Initial kernel scaffold; baseline (speedup 1.0000x reference)
import jax
import jax.numpy as jnp
from jax import lax
from jax.experimental import pallas as pl
from jax.experimental.pallas import tpu as pltpu
from jax.experimental.pallas import tpu_sc as plsc

Q = 1_000_000
D = 16
B = 16384
NC = 2
NS = 16
NW = NC * NS
R = Q // NW
L = 16
CAPB = 1536
NR = CAPB // 128


def _sc_body(bf_hbm, idx_hbm, out_hbm,
             idx_v, gl, pv, keep, fgl, fpl, tag, fg2, fp2, rows, sem, sem2):
  wid = lax.axis_index("s") * NC + lax.axis_index("c")
  base = (wid * R).astype(jnp.int32)
  iota = lax.iota(jnp.int32, L)

  pltpu.sync_copy(idx_hbm, idx_v)

  def fbody(j, n):
    v = idx_v[pl.ds(j * L, L)]
    m = (v >= base) & (v < base + R)
    plsc.store_compressed(gl.at[pl.ds(n, L)], v, mask=m)
    plsc.store_compressed(pv.at[pl.ds(n, L)], iota + j * L, mask=m)
    return n + jnp.max(plsc.all_reduce_population_count(m))

  n = lax.fori_loop(0, B // L, fbody, jnp.int32(0))
  nch = (n + L - 1) // L

  def cdbody(c, _):
    off = c * L
    v = gl[pl.ds(off, L)]
    lim = jnp.minimum(n - off, L)
    k = iota >= lim
    for s in range(1, L):
      rot = jnp.take(v, (iota + s) & (L - 1), mode="promise_in_bounds")
      k = k | ((v == rot) & (iota + s < lim))
    km = ~k
    keep[pl.ds(off, L)] = km.astype(jnp.int32)
    plsc.store_scatter(tag, [v - base], iota + off, mask=km)
    return 0

  lax.fori_loop(0, nch, cdbody, 0)

  def ebody(c, n2):
    off = c * L
    v = gl[pl.ds(off, L)]
    km = keep[pl.ds(off, L)] != 0
    t = plsc.load_gather(tag, [v - base], mask=km)
    win = km & (t == iota + off)
    plsc.store_compressed(fgl.at[pl.ds(n2, L)], v, mask=win)
    plsc.store_compressed(fpl.at[pl.ds(n2, L)], pv[pl.ds(off, L)], mask=win)
    return n2 + jnp.max(plsc.all_reduce_population_count(win))

  n2 = lax.fori_loop(0, nch, ebody, jnp.int32(0))

  @pl.when(n2 > 0)
  def _pad():
    last = n2 - 1
    lane = iota * 0 + (last & (L - 1))
    lv = jnp.take(fgl[pl.ds((last // L) * L, L)], lane,
                  mode="promise_in_bounds")
    lp = jnp.take(fpl[pl.ds((last // L) * L, L)], lane,
                  mode="promise_in_bounds")
    for t in range(128 // L + 1):
      fgl[pl.ds(n2 + t * L, L)] = lv
      fpl[pl.ds(n2 + t * L, L)] = lp

  nf = (n2 + 127) // 128

  def cpbody(c, _):
    pltpu.sync_copy(fgl.at[pl.ds(c * 128, 128)], fg2.at[c])
    pltpu.sync_copy(fpl.at[pl.ds(c * 128, 128)], fp2.at[c])
    return 0

  lax.fori_loop(0, nf, cpbody, 0)

  def gfire(c, _):
    pltpu.async_copy(bf_hbm.at[fp2.at[c]], rows.at[pl.ds(c * 128, 128)], sem)
    return 0

  def gdrain(c, _):
    pltpu.make_async_copy(
        bf_hbm.at[fp2.at[c]], rows.at[pl.ds(c * 128, 128)], sem).wait()
    return 0

  lax.fori_loop(0, nf, gfire, 0)
  lax.fori_loop(0, nf, gdrain, 0)

  def sfire(c, _):
    pltpu.async_copy(rows.at[pl.ds(c * 128, 128)], out_hbm.at[fg2.at[c]], sem2)
    return 0

  def sdrain(c, _):
    pltpu.make_async_copy(
        rows.at[pl.ds(c * 128, 128)], out_hbm.at[fg2.at[c]], sem2).wait()
    return 0

  lax.fori_loop(0, nf, sfire, 0)
  lax.fori_loop(0, nf, sdrain, 0)


_mesh = plsc.VectorSubcoreMesh(
    core_axis_name="c", subcore_axis_name="s", num_cores=NC, num_subcores=NS)

_sc_update = pl.kernel(
    _sc_body,
    out_type=(),
    mesh=_mesh,
    scratch_types=[
        pltpu.VMEM((B,), jnp.int32),
        pltpu.VMEM((CAPB,), jnp.int32),
        pltpu.VMEM((CAPB,), jnp.int32),
        pltpu.VMEM((CAPB,), jnp.int32),
        pltpu.VMEM((CAPB,), jnp.int32),
        pltpu.VMEM((CAPB,), jnp.int32),
        pltpu.VMEM((R,), jnp.int32),
        pltpu.VMEM((NR, 128), jnp.int32),
        pltpu.VMEM((NR, 128), jnp.int32),
        pltpu.VMEM((CAPB, D), jnp.float32),
        pltpu.SemaphoreType.DMA,
        pltpu.SemaphoreType.DMA,
    ],
    name="key_memory_scatter",
)


@jax.jit
def kernel(batch_features, batch_indices, features):
  out_ref = jax.new_ref(features)
  _sc_update(batch_features, batch_indices.astype(jnp.int32), out_ref)
  return out_ref[...]


# speedup vs baseline: 2.1914x; 2.1914x over previous
import jax
import jax.numpy as jnp
from jax import lax
from jax.experimental import pallas as pl
from jax.experimental.pallas import tpu as pltpu
from jax.experimental.pallas import tpu_sc as plsc

Q = 1_000_000
D = 16
B = 16384
NC = 2
NS = 16
NW = NC * NS
R = Q // NW
L = 16
CAPB = 1536
NR = CAPB // 128


def _sc_body(bf_hbm, idx_hbm, out_hbm,
             idx_v, gl, pv, keep, tag, fg2, fp2, rows, sem, sem2):
  wid = lax.axis_index("s") * NC + lax.axis_index("c")
  base = (wid * R).astype(jnp.int32)
  iota = lax.iota(jnp.int32, L)

  pltpu.sync_copy(idx_hbm, idx_v)

  def fbody(j, n):
    v = idx_v[pl.ds(j * L, L)]
    m = (v >= base) & (v < base + R)
    c = plsc.cumsum(m.astype(jnp.int32))
    dst = n + c - 1
    plsc.store_scatter(gl, [dst], v, mask=m)
    plsc.store_scatter(pv, [dst], iota + j * L, mask=m)
    return n + jnp.max(c)

  n = lax.fori_loop(0, B // L, fbody, jnp.int32(0))
  nch = (n + L - 1) // L

  def cdbody(c, _):
    off = c * L
    v = gl[pl.ds(off, L)]
    valid = iota < n - off
    _, lastm = plsc.scan_count(v, valid)
    km = lastm & valid
    keep[pl.ds(off, L)] = km.astype(jnp.int32)
    plsc.store_scatter(tag, [v - base], iota + off, mask=km)
    return 0

  lax.fori_loop(0, nch, cdbody, 0)

  def ebody(c, n2):
    off = c * L
    v = gl[pl.ds(off, L)]
    km = keep[pl.ds(off, L)] != 0
    t = plsc.load_gather(tag, [v - base], mask=km)
    win = km & (t == iota + off)
    c2 = plsc.cumsum(win.astype(jnp.int32))
    dst = n2 + c2 - 1
    plsc.store_scatter(fg2, [dst // 128, dst % 128], v, mask=win)
    plsc.store_scatter(fp2, [dst // 128, dst % 128], pv[pl.ds(off, L)],
                       mask=win)
    return n2 + jnp.max(c2)

  n2 = lax.fori_loop(0, nch, ebody, jnp.int32(0))

  @pl.when(n2 > 0)
  def _pad():
    last = iota * 0 + (n2 - 1)
    lv = plsc.load_gather(fg2, [last // 128, last % 128])
    lp = plsc.load_gather(fp2, [last // 128, last % 128])
    for t in range(128 // L + 1):
      dst = n2 + t * L + iota
      plsc.store_scatter(fg2, [dst // 128, dst % 128], lv)
      plsc.store_scatter(fp2, [dst // 128, dst % 128], lp)

  nf = (n2 + 127) // 128

  def gfire(c, _):
    pltpu.async_copy(bf_hbm.at[fp2.at[c]], rows.at[pl.ds(c * 128, 128)], sem)
    return 0

  def gdrain(c, _):
    pltpu.make_async_copy(
        bf_hbm.at[fp2.at[c]], rows.at[pl.ds(c * 128, 128)], sem).wait()
    return 0

  lax.fori_loop(0, nf, gfire, 0)
  lax.fori_loop(0, nf, gdrain, 0)

  def sfire(c, _):
    pltpu.async_copy(rows.at[pl.ds(c * 128, 128)], out_hbm.at[fg2.at[c]], sem2)
    return 0

  def sdrain(c, _):
    pltpu.make_async_copy(
        rows.at[pl.ds(c * 128, 128)], out_hbm.at[fg2.at[c]], sem2).wait()
    return 0

  lax.fori_loop(0, nf, sfire, 0)
  lax.fori_loop(0, nf, sdrain, 0)


_mesh = plsc.VectorSubcoreMesh(
    core_axis_name="c", subcore_axis_name="s", num_cores=NC, num_subcores=NS)

_sc_update = pl.kernel(
    _sc_body,
    out_type=(),
    mesh=_mesh,
    compiler_params=pltpu.CompilerParams(
        use_tc_tiling_on_sc=False, needs_layout_passes=False),
    scratch_types=[
        pltpu.VMEM((B,), jnp.int32),
        pltpu.VMEM((CAPB,), jnp.int32),
        pltpu.VMEM((CAPB,), jnp.int32),
        pltpu.VMEM((CAPB,), jnp.int32),
        pltpu.VMEM((R,), jnp.int32),
        pltpu.VMEM((NR, 128), jnp.int32),
        pltpu.VMEM((NR, 128), jnp.int32),
        pltpu.VMEM((CAPB, D), jnp.float32),
        pltpu.SemaphoreType.DMA,
        pltpu.SemaphoreType.DMA,
    ],
    name="key_memory_scatter",
)


@jax.jit
def kernel(batch_features, batch_indices, features):
  out_ref = jax.new_ref(features)
  _sc_update(batch_features, batch_indices.astype(jnp.int32), out_ref)
  return out_ref[...]
